# baseline (speedup 1.0000x reference)
import jax
import jax.numpy as jnp
from jax.experimental import pallas as pl
from jax.experimental.pallas import tpu as pltpu
from jax.experimental.pallas import tpu_sc as plsc

_LANES = 16


def _sc_add(x2d, pos_table, S):
    R, D = x2d.shape
    BR = 8
    BC = D

    mesh = plsc.VectorSubcoreMesh(core_axis_name="core",
                                  subcore_axis_name="subcore")

    @pl.kernel(out_type=jax.ShapeDtypeStruct((R, D), x2d.dtype), mesh=mesh)
    def run(x_hbm, pe_hbm, o_hbm):
        def body(x_vmem, pe_vmem, o_vmem):
            @pl.loop(0, BR)
            def _(r):
                @pl.loop(0, BC, step=_LANES)
                def _(c):
                    slc = (pl.ds(r, 1), pl.ds(c, _LANES))
                    o_vmem.at[*slc][...] = (
                        x_vmem.at[*slc][...] + pe_vmem.at[*slc][...]
                    )

        pltpu.emit_pipeline(
            body,
            grid=(R // BR,),
            in_specs=[
                pl.BlockSpec((BR, BC), lambda i: (i, 0)),
                pl.BlockSpec((BR, BC), lambda i: (i % (S // BR), 0)),
            ],
            out_specs=[pl.BlockSpec((BR, BC), lambda i: (i, 0))],
            core_axis_name=("core", "subcore"),
            dimension_semantics=(pltpu.PARALLEL,),
        )(x_hbm, pe_hbm, o_hbm)

    return run(x2d, pos_table)


def kernel(x, pos_table):
    B, S, D = x.shape
    x2d = x.reshape(B * S, D)
    out = _sc_add(x2d, pos_table, S)
    return out.reshape(B, S, D)


# speedup vs baseline: 1.6963x; 1.6963x over previous
import jax
import jax.numpy as jnp
from jax.experimental import pallas as pl
from jax.experimental.pallas import tpu as pltpu
from jax.experimental.pallas import tpu_sc as plsc

_LANES = 16


def _tc_body(x_ref, pe_ref, o_ref):
    o_ref[...] = x_ref[...] + pe_ref[...]


def _tc_add(x, pos_table, n_batch):
    B, S, D = x.shape
    BLK = 2048
    return pl.pallas_call(
        _tc_body,
        grid=(S // BLK, n_batch),
        in_specs=[
            pl.BlockSpec((1, BLK, D), lambda i, j: (j, i, 0)),
            pl.BlockSpec((BLK, D), lambda i, j: (i, 0)),
        ],
        out_specs=pl.BlockSpec((1, BLK, D), lambda i, j: (j, i, 0)),
        out_shape=jax.ShapeDtypeStruct((n_batch, S, D), x.dtype),
        compiler_params=pltpu.CompilerParams(
            dimension_semantics=("parallel", "arbitrary"),
        ),
    )(x, pos_table)


def _sc_add(x, pos_table, batch_lo):
    B, S, D = x.shape
    n_batch = B - batch_lo
    R = n_batch * S
    BR = 8

    mesh = plsc.VectorSubcoreMesh(core_axis_name="core",
                                  subcore_axis_name="subcore")

    @pl.kernel(out_type=jax.ShapeDtypeStruct((R, D), x.dtype), mesh=mesh)
    def run(x_hbm, pe_hbm, o_hbm):
        def body(x_vmem, pe_vmem, o_vmem):
            @pl.loop(0, BR)
            def _(r):
                @pl.loop(0, D, step=_LANES)
                def _(c):
                    slc = (pl.ds(r, 1), pl.ds(c, _LANES))
                    o_vmem.at[*slc][...] = (
                        x_vmem.at[*slc][...] + pe_vmem.at[*slc][...]
                    )

        row0 = batch_lo * S // BR
        pltpu.emit_pipeline(
            body,
            grid=(R // BR,),
            in_specs=[
                pl.BlockSpec((BR, D), lambda i: (row0 + i, 0)),
                pl.BlockSpec((BR, D), lambda i: (i % (S // BR), 0)),
            ],
            out_specs=[pl.BlockSpec((BR, D), lambda i: (i, 0))],
            core_axis_name=("core", "subcore"),
            dimension_semantics=(pltpu.PARALLEL,),
        )(x_hbm, pe_hbm, o_hbm)

    x2d = x.reshape(B * S, D)
    return run(x2d, pos_table).reshape(n_batch, S, D)


def kernel(x, pos_table):
    B, S, D = x.shape
    split = B - 1
    out_tc = _tc_add(x, pos_table, split)
    out_sc = _sc_add(x, pos_table, split)
    return jnp.concatenate([out_tc, out_sc], axis=0)


# speedup vs baseline: 4.1220x; 2.4300x over previous
import jax
import jax.numpy as jnp
from jax.experimental import pallas as pl
from jax.experimental.pallas import tpu as pltpu


def _add_body(x_ref, pe_ref, o_ref):
    o_ref[...] = x_ref[...] + pe_ref[...]


def kernel(x, pos_table):
    B, S, D = x.shape
    BLK = 2048

    out = pl.pallas_call(
        _add_body,
        grid=(S // BLK, B),
        in_specs=[
            pl.BlockSpec((1, BLK, D), lambda i, j: (j, i, 0)),
            pl.BlockSpec((BLK, D), lambda i, j: (i, 0)),
        ],
        out_specs=pl.BlockSpec((1, BLK, D), lambda i, j: (j, i, 0)),
        out_shape=jax.ShapeDtypeStruct((B, S, D), x.dtype),
        compiler_params=pltpu.CompilerParams(
            dimension_semantics=("parallel", "parallel"),
        ),
    )(x, pos_table)
    return out


# speedup vs baseline: 4.1331x; 1.0027x over previous
import jax
import jax.numpy as jnp
from jax.experimental import pallas as pl
from jax.experimental.pallas import tpu as pltpu


def _add_body(x_ref, pe_ref, o_ref):
    o_ref[...] = x_ref[...] + pe_ref[...]


def kernel(x, pos_table):
    B, S, D = x.shape
    BLK = next(b for b in (2048, 1024, 512, 256, 128, 8, 1) if S % b == 0)

    out = pl.pallas_call(
        _add_body,
        grid=(S // BLK, B),
        in_specs=[
            pl.BlockSpec((1, BLK, D), lambda i, j: (j, i, 0)),
            pl.BlockSpec((BLK, D), lambda i, j: (i, 0)),
        ],
        out_specs=pl.BlockSpec((1, BLK, D), lambda i, j: (j, i, 0)),
        out_shape=jax.ShapeDtypeStruct((B, S, D), x.dtype),
        compiler_params=pltpu.CompilerParams(
            dimension_semantics=("parallel", "parallel"),
        ),
    )(x, pos_table)
    return out
